# baseline (speedup 1.0000x reference)
import functools

import jax
import jax.numpy as jnp
from jax import lax
from jax.experimental import pallas as pl
from jax.experimental.pallas import tpu as pltpu
from jax.experimental.pallas import tpu_sc as plsc

_N = 20000
_L = 16
_C1 = 1264
_MAX_DET = 100
_KCAP = 112
_OUTW = 512
_IOU_THRESH = 0.5
_SCORE_THRESH = 0.05
_NEG = -1e9


def _thresh(v):
    return jnp.where(v > _SCORE_THRESH, v,
                     jnp.where(v < -1e30, -jnp.inf, jnp.float32(_NEG)))


def _sc_body(b_hbm, sc_hbm, out_hbm,
             s_ref, b_ref, l1_ref, l2_ref, l3_ref,
             kx1, ky1, kx2, ky2, ka, out_ref, fil_ref):
    tile0 = (lax.axis_index("c") == 0) & (lax.axis_index("s") == 0)

    @pl.when(tile0)
    def _():
        iota = lax.broadcasted_iota(jnp.int32, (_L,), 0)
        lane0 = iota == 0
        out5 = iota < 5
        ninf = jnp.full((_L,), -jnp.inf, jnp.float32)

        pltpu.sync_copy(b_hbm, b_ref)
        pltpu.sync_copy(sc_hbm, s_ref.at[pl.ds(0, _N)])

        for t in range((_C1 * _L - _N) // _L):
            s_ref[pl.ds(_N + t * _L, _L)] = ninf
        l1_ref[pl.ds(_C1, _L)] = ninf

        def l1_body(k, _):
            base = k * (_L * _L)
            acc = plsc.load_gather(s_ref, [iota * _L + base])
            for t in range(1, _L):
                acc = jnp.maximum(
                    acc, plsc.load_gather(s_ref, [iota * _L + (base + t)]))
            l1_ref[pl.ds(k * _L, _L)] = _thresh(acc)
            return 0
        lax.fori_loop(0, _C1 // _L, l1_body, 0)

        def l2_body(k, _):
            base = k * (_L * _L)
            acc = plsc.load_gather(l1_ref, [iota * _L + base])
            for t in range(1, _L):
                acc = jnp.maximum(
                    acc, plsc.load_gather(l1_ref, [iota * _L + (base + t)]))
            l2_ref[pl.ds(k * _L, _L)] = acc
            return 0
        lax.fori_loop(0, 80 // _L, l2_body, 0)

        l3 = ninf
        for q in range(80 // _L):
            mq = jnp.max(l2_ref[pl.ds(q * _L, _L)])
            l3 = jnp.where(iota == q, jnp.full((_L,), mq), l3)
        l3_ref[...] = l3

        def cond(st):
            kc, fc, processed = st
            return (kc < _MAX_DET) & (processed < _N)

        def body(st):
            kc, fc, processed = st
            l3v = l3_ref[...]
            m = jnp.max(l3v)
            m_v = jnp.full((_L,), m)
            q_v = plsc.all_reduce_ffs(l3v == m_v)
            l2v = plsc.load_gather(l2_ref, [q_v * _L + iota])
            g_v = q_v * _L + plsc.all_reduce_ffs(l2v == m_v)
            l1v = plsc.load_gather(l1_ref, [g_v * _L + iota])
            c_v = g_v * _L + plsc.all_reduce_ffs(l1v == m_v)
            sv = _thresh(plsc.load_gather(s_ref, [c_v * _L + iota]))
            j_v = c_v * _L + plsc.all_reduce_ffs(sv == m_v)

            j4 = j_v * 4
            bx1 = plsc.load_gather(b_ref, [j4])
            by1 = plsc.load_gather(b_ref, [j4 + 1])
            bx2 = plsc.load_gather(b_ref, [j4 + 2])
            by2 = plsc.load_gather(b_ref, [j4 + 3])
            area = (jnp.maximum(bx2 - bx1, 0.0)
                    * jnp.maximum(by2 - by1, 0.0))

            acc = jnp.full((_L,), -1.0, jnp.float32)
            for kk in range(_KCAP // _L):
                sl = pl.ds(kk * _L, _L)
                xx1 = jnp.maximum(kx1[sl], bx1)
                yy1 = jnp.maximum(ky1[sl], by1)
                xx2 = jnp.minimum(kx2[sl], bx2)
                yy2 = jnp.minimum(ky2[sl], by2)
                inter = (jnp.maximum(xx2 - xx1, 0.0)
                         * jnp.maximum(yy2 - yy1, 0.0))
                iou = inter / (ka[sl] + area - inter + 1e-9)
                lanemask = (iota + kk * _L) < kc
                acc = jnp.maximum(acc, jnp.where(lanemask, iou, -1.0))
            suppressed = jnp.max(acc) > _IOU_THRESH
            valid = m > -1e8
            keep_it = valid & jnp.logical_not(suppressed)

            row = bx1
            row = jnp.where(iota == 1, by1, row)
            row = jnp.where(iota == 2, bx2, row)
            row = jnp.where(iota == 3, by2, row)
            row = jnp.where(iota == 4, m_v, row)

            @pl.when(keep_it)
            def _():
                kc_v = jnp.full((_L,), kc)
                plsc.store_scatter(kx1, [kc_v], bx1, mask=lane0)
                plsc.store_scatter(ky1, [kc_v], by1, mask=lane0)
                plsc.store_scatter(kx2, [kc_v], bx2, mask=lane0)
                plsc.store_scatter(ky2, [kc_v], by2, mask=lane0)
                plsc.store_scatter(ka, [kc_v], area, mask=lane0)
                plsc.store_scatter(out_ref, [kc_v * 5 + iota], row, mask=out5)

            filler_slot = jnp.logical_not(keep_it) & (fc < _MAX_DET)

            @pl.when(filler_slot)
            def _():
                fc_v = jnp.full((_L,), fc)
                plsc.store_scatter(fil_ref, [fc_v * 5 + iota], row, mask=out5)

            plsc.store_scatter(s_ref, [j_v], ninf, mask=lane0)
            sv2 = plsc.load_gather(s_ref, [c_v * _L + iota])
            plsc.store_scatter(
                l1_ref, [c_v],
                _thresh(jnp.full((_L,), jnp.max(sv2))), mask=lane0)
            l1v2 = plsc.load_gather(l1_ref, [g_v * _L + iota])
            plsc.store_scatter(l2_ref, [g_v],
                               jnp.full((_L,), jnp.max(l1v2)), mask=lane0)
            l2v2 = plsc.load_gather(l2_ref, [q_v * _L + iota])
            plsc.store_scatter(l3_ref, [q_v],
                               jnp.full((_L,), jnp.max(l2v2)), mask=lane0)

            kc = jnp.where(keep_it, kc + 1, kc)
            fc = jnp.where(filler_slot, fc + 1, fc)
            return (kc, fc, processed + 1)

        kc, fc, _ = lax.while_loop(
            cond, body, (jnp.int32(0), jnp.int32(0), jnp.int32(0)))

        def fcond(i):
            return i < _MAX_DET

        def fbody(i):
            src = jnp.full((_L,), i - kc) * 5 + iota
            dst = jnp.full((_L,), i) * 5 + iota
            v = plsc.load_gather(fil_ref, [src])
            plsc.store_scatter(out_ref, [dst], v, mask=out5)
            return i + 1

        lax.while_loop(fcond, fbody, kc)

        pltpu.sync_copy(out_ref, out_hbm)


_sc_nms = functools.partial(
    pl.kernel,
    out_type=jax.ShapeDtypeStruct((_OUTW,), jnp.float32),
    mesh=plsc.VectorSubcoreMesh(core_axis_name="c", subcore_axis_name="s"),
    compiler_params=pltpu.CompilerParams(needs_layout_passes=False),
    scratch_types=[
        pltpu.VMEM((_C1 * _L,), jnp.float32),
        pltpu.VMEM((_N * 4,), jnp.float32),
        pltpu.VMEM((_C1 + _L,), jnp.float32),
        pltpu.VMEM((80,), jnp.float32),
        pltpu.VMEM((_L,), jnp.float32),
        pltpu.VMEM((_KCAP,), jnp.float32),
        pltpu.VMEM((_KCAP,), jnp.float32),
        pltpu.VMEM((_KCAP,), jnp.float32),
        pltpu.VMEM((_KCAP,), jnp.float32),
        pltpu.VMEM((_KCAP,), jnp.float32),
        pltpu.VMEM((_OUTW,), jnp.float32),
        pltpu.VMEM((_OUTW,), jnp.float32),
    ],
)(_sc_body)


def kernel(boxes, scores):
    out = _sc_nms(boxes.reshape(-1), scores)
    return out[:_MAX_DET * 5].reshape(_MAX_DET, 5)


# speedup vs baseline: 1.3871x; 1.3871x over previous
import functools

import jax
import jax.numpy as jnp
from jax import lax
from jax.experimental import pallas as pl
from jax.experimental.pallas import tpu as pltpu
from jax.experimental.pallas import tpu_sc as plsc

_N = 20000
_L = 16
_C1 = 1264
_MAX_DET = 100
_KCAP = 112
_OUTW = 512
_IOU_THRESH = 0.5
_SCORE_THRESH = 0.05
_NEG = -1e9


def _thresh(v):
    return jnp.where(v > _SCORE_THRESH, v,
                     jnp.where(v < -1e30, -jnp.inf, jnp.float32(_NEG)))


def _sc_body(x1_hbm, y1_hbm, x2_hbm, y2_hbm, sc_hbm, out_hbm,
             s_ref, x1_ref, y1_ref, x2_ref, y2_ref,
             l1_ref, l2_ref, l3_ref,
             kx1, ky1, kx2, ky2, ka, out_ref, fil_ref, dma_sem):
    tile0 = (lax.axis_index("c") == 0) & (lax.axis_index("s") == 0)

    @pl.when(tile0)
    def _():
        iota = lax.broadcasted_iota(jnp.int32, (_L,), 0)
        lane0 = iota == 0
        out5 = iota < 5
        ninf = jnp.full((_L,), -jnp.inf, jnp.float32)

        cps = [pltpu.async_copy(src, dst, dma_sem)
               for src, dst in ((x1_hbm, x1_ref), (y1_hbm, y1_ref),
                                (x2_hbm, x2_ref), (y2_hbm, y2_ref))]
        pltpu.sync_copy(sc_hbm, s_ref.at[pl.ds(0, _N)])

        for t in range((_C1 * _L - _N) // _L):
            s_ref[pl.ds(_N + t * _L, _L)] = ninf
        l1_ref[pl.ds(_C1, _L)] = ninf

        def l1_body(k, _):
            base = k * (_L * _L)
            acc = plsc.load_gather(s_ref, [iota * _L + base])
            for t in range(1, _L):
                acc = jnp.maximum(
                    acc, plsc.load_gather(s_ref, [iota * _L + (base + t)]))
            l1_ref[pl.ds(k * _L, _L)] = _thresh(acc)
            return 0
        lax.fori_loop(0, _C1 // _L, l1_body, 0)

        def l2_body(k, _):
            base = k * (_L * _L)
            acc = plsc.load_gather(l1_ref, [iota * _L + base])
            for t in range(1, _L):
                acc = jnp.maximum(
                    acc, plsc.load_gather(l1_ref, [iota * _L + (base + t)]))
            l2_ref[pl.ds(k * _L, _L)] = acc
            return 0
        lax.fori_loop(0, 80 // _L, l2_body, 0)

        l3 = ninf
        for q in range(80 // _L):
            mq = jnp.max(l2_ref[pl.ds(q * _L, _L)])
            l3 = jnp.where(iota == q, jnp.full((_L,), mq), l3)
        l3_ref[...] = l3

        for cp in cps:
            cp.wait()

        def cond(st):
            kc, fc, processed = st
            return (kc < _MAX_DET) & (processed < _N)

        def body(st):
            kc, fc, processed = st
            l3v = l3_ref[...]
            m = jnp.max(l3v)
            m_v = jnp.full((_L,), m)
            q_v = plsc.all_reduce_ffs(l3v == m_v)
            l2v = plsc.load_gather(l2_ref, [q_v * _L + iota])
            g_v = q_v * _L + plsc.all_reduce_ffs(l2v == m_v)
            l1v = plsc.load_gather(l1_ref, [g_v * _L + iota])
            c_v = g_v * _L + plsc.all_reduce_ffs(l1v == m_v)
            sv = _thresh(plsc.load_gather(s_ref, [c_v * _L + iota]))
            j_v = c_v * _L + plsc.all_reduce_ffs(sv == m_v)

            bx1 = plsc.load_gather(x1_ref, [j_v])
            by1 = plsc.load_gather(y1_ref, [j_v])
            bx2 = plsc.load_gather(x2_ref, [j_v])
            by2 = plsc.load_gather(y2_ref, [j_v])
            area = (jnp.maximum(bx2 - bx1, 0.0)
                    * jnp.maximum(by2 - by1, 0.0))

            acc = jnp.full((_L,), -1.0, jnp.float32)
            for kk in range(_KCAP // _L):
                sl = pl.ds(kk * _L, _L)
                xx1 = jnp.maximum(kx1[sl], bx1)
                yy1 = jnp.maximum(ky1[sl], by1)
                xx2 = jnp.minimum(kx2[sl], bx2)
                yy2 = jnp.minimum(ky2[sl], by2)
                inter = (jnp.maximum(xx2 - xx1, 0.0)
                         * jnp.maximum(yy2 - yy1, 0.0))
                iou = inter / (ka[sl] + area - inter + 1e-9)
                lanemask = (iota + kk * _L) < kc
                acc = jnp.maximum(acc, jnp.where(lanemask, iou, -1.0))
            suppressed = jnp.max(acc) > _IOU_THRESH
            valid = m > -1e8
            keep_it = valid & jnp.logical_not(suppressed)

            row = bx1
            row = jnp.where(iota == 1, by1, row)
            row = jnp.where(iota == 2, bx2, row)
            row = jnp.where(iota == 3, by2, row)
            row = jnp.where(iota == 4, m_v, row)

            @pl.when(keep_it)
            def _():
                kc_v = jnp.full((_L,), kc)
                plsc.store_scatter(kx1, [kc_v], bx1, mask=lane0)
                plsc.store_scatter(ky1, [kc_v], by1, mask=lane0)
                plsc.store_scatter(kx2, [kc_v], bx2, mask=lane0)
                plsc.store_scatter(ky2, [kc_v], by2, mask=lane0)
                plsc.store_scatter(ka, [kc_v], area, mask=lane0)
                plsc.store_scatter(out_ref, [kc_v * 5 + iota], row, mask=out5)

            filler_slot = jnp.logical_not(keep_it) & (fc < _MAX_DET)

            @pl.when(filler_slot)
            def _():
                fc_v = jnp.full((_L,), fc)
                plsc.store_scatter(fil_ref, [fc_v * 5 + iota], row, mask=out5)

            plsc.store_scatter(s_ref, [j_v], ninf, mask=lane0)
            sv2 = plsc.load_gather(s_ref, [c_v * _L + iota])
            plsc.store_scatter(
                l1_ref, [c_v],
                _thresh(jnp.full((_L,), jnp.max(sv2))), mask=lane0)
            l1v2 = plsc.load_gather(l1_ref, [g_v * _L + iota])
            plsc.store_scatter(l2_ref, [g_v],
                               jnp.full((_L,), jnp.max(l1v2)), mask=lane0)
            l2v2 = plsc.load_gather(l2_ref, [q_v * _L + iota])
            plsc.store_scatter(l3_ref, [q_v],
                               jnp.full((_L,), jnp.max(l2v2)), mask=lane0)

            kc = jnp.where(keep_it, kc + 1, kc)
            fc = jnp.where(filler_slot, fc + 1, fc)
            return (kc, fc, processed + 1)

        kc, fc, _ = lax.while_loop(
            cond, body, (jnp.int32(0), jnp.int32(0), jnp.int32(0)))

        def fcond(i):
            return i < _MAX_DET

        def fbody(i):
            src = jnp.full((_L,), i - kc) * 5 + iota
            dst = jnp.full((_L,), i) * 5 + iota
            v = plsc.load_gather(fil_ref, [src])
            plsc.store_scatter(out_ref, [dst], v, mask=out5)
            return i + 1

        lax.while_loop(fcond, fbody, kc)

        pltpu.sync_copy(out_ref, out_hbm)


_sc_nms = functools.partial(
    pl.kernel,
    out_type=jax.ShapeDtypeStruct((_OUTW,), jnp.float32),
    mesh=plsc.VectorSubcoreMesh(core_axis_name="c", subcore_axis_name="s"),
    compiler_params=pltpu.CompilerParams(needs_layout_passes=False),
    scratch_types=[
        pltpu.VMEM((_C1 * _L,), jnp.float32),
        pltpu.VMEM((_N,), jnp.float32),
        pltpu.VMEM((_N,), jnp.float32),
        pltpu.VMEM((_N,), jnp.float32),
        pltpu.VMEM((_N,), jnp.float32),
        pltpu.VMEM((_C1 + _L,), jnp.float32),
        pltpu.VMEM((80,), jnp.float32),
        pltpu.VMEM((_L,), jnp.float32),
        pltpu.VMEM((_KCAP,), jnp.float32),
        pltpu.VMEM((_KCAP,), jnp.float32),
        pltpu.VMEM((_KCAP,), jnp.float32),
        pltpu.VMEM((_KCAP,), jnp.float32),
        pltpu.VMEM((_KCAP,), jnp.float32),
        pltpu.VMEM((_OUTW,), jnp.float32),
        pltpu.VMEM((_OUTW,), jnp.float32),
        pltpu.SemaphoreType.DMA,
    ],
)(_sc_body)


def kernel(boxes, scores):
    out = _sc_nms(boxes[:, 0], boxes[:, 1], boxes[:, 2], boxes[:, 3], scores)
    return out[:_MAX_DET * 5].reshape(_MAX_DET, 5)


# speedup vs baseline: 1.4287x; 1.0299x over previous
import functools

import jax
import jax.numpy as jnp
from jax import lax
from jax.experimental import pallas as pl
from jax.experimental.pallas import tpu as pltpu
from jax.experimental.pallas import tpu_sc as plsc

_N = 20000
_L = 16
_C1 = 1264
_MAX_DET = 100
_KCAP = 112
_OUTW = 512
_IOU_THRESH = 0.5
_SCORE_THRESH = 0.05
_NEG = -1e9


def _thresh(v):
    return jnp.where(v > _SCORE_THRESH, v,
                     jnp.where(v < -1e30, -jnp.inf, jnp.float32(_NEG)))


def _sc_body(x1_hbm, y1_hbm, x2_hbm, y2_hbm, sc_hbm, out_hbm,
             s_ref, x1_ref, y1_ref, x2_ref, y2_ref,
             l1_ref, l2_ref, l3_ref,
             kx1, ky1, kx2, ky2, ka, out_ref, fil_ref, dma_sem):
    tile0 = (lax.axis_index("c") == 0) & (lax.axis_index("s") == 0)

    @pl.when(tile0)
    def _():
        iota = lax.broadcasted_iota(jnp.int32, (_L,), 0)
        lane0 = iota == 0
        out5 = iota < 5
        ninf = jnp.full((_L,), -jnp.inf, jnp.float32)

        cps = [pltpu.async_copy(src, dst, dma_sem)
               for src, dst in ((x1_hbm, x1_ref), (y1_hbm, y1_ref),
                                (x2_hbm, x2_ref), (y2_hbm, y2_ref))]
        pltpu.sync_copy(sc_hbm, s_ref.at[pl.ds(0, _N)])

        for t in range((_C1 * _L - _N) // _L):
            s_ref[pl.ds(_N + t * _L, _L)] = ninf
        l1_ref[pl.ds(_C1, _L)] = ninf

        def l1_body(k, _):
            base = k * (_L * _L)
            acc = plsc.load_gather(s_ref, [base + iota * _L + iota])
            for t in range(1, _L):
                acc = jnp.maximum(
                    acc,
                    plsc.load_gather(
                        s_ref, [base + iota * _L + ((iota + t) & (_L - 1))]))
            l1_ref[pl.ds(k * _L, _L)] = _thresh(acc)
            return 0
        lax.fori_loop(0, _C1 // _L, l1_body, 0)

        def l2_body(k, _):
            base = k * (_L * _L)
            acc = plsc.load_gather(l1_ref, [base + iota * _L + iota])
            for t in range(1, _L):
                acc = jnp.maximum(
                    acc,
                    plsc.load_gather(
                        l1_ref, [base + iota * _L + ((iota + t) & (_L - 1))]))
            l2_ref[pl.ds(k * _L, _L)] = acc
            return 0
        lax.fori_loop(0, 80 // _L, l2_body, 0)

        l3 = ninf
        for q in range(80 // _L):
            mq = jnp.max(l2_ref[pl.ds(q * _L, _L)])
            l3 = jnp.where(iota == q, jnp.full((_L,), mq), l3)
        l3_ref[...] = l3

        for cp in cps:
            cp.wait()

        def cond(st):
            kc, fc, processed = st
            return (kc < _MAX_DET) & (processed < _N)

        def body(st):
            kc, fc, processed = st
            l3v = l3_ref[...]
            m = jnp.max(l3v)
            m_v = jnp.full((_L,), m)
            q_v = plsc.all_reduce_ffs(l3v == m_v)
            l2v = plsc.load_gather(l2_ref, [q_v * _L + iota])
            g_v = q_v * _L + plsc.all_reduce_ffs(l2v == m_v)
            l1v = plsc.load_gather(l1_ref, [g_v * _L + iota])
            c_v = g_v * _L + plsc.all_reduce_ffs(l1v == m_v)
            sv = _thresh(plsc.load_gather(s_ref, [c_v * _L + iota]))
            j_v = c_v * _L + plsc.all_reduce_ffs(sv == m_v)

            bx1 = plsc.load_gather(x1_ref, [j_v])
            by1 = plsc.load_gather(y1_ref, [j_v])
            bx2 = plsc.load_gather(x2_ref, [j_v])
            by2 = plsc.load_gather(y2_ref, [j_v])
            area = (jnp.maximum(bx2 - bx1, 0.0)
                    * jnp.maximum(by2 - by1, 0.0))

            def iou_blk(kk, acc):
                sl = pl.ds(kk * _L, _L)
                xx1 = jnp.maximum(kx1[sl], bx1)
                yy1 = jnp.maximum(ky1[sl], by1)
                xx2 = jnp.minimum(kx2[sl], bx2)
                yy2 = jnp.minimum(ky2[sl], by2)
                inter = (jnp.maximum(xx2 - xx1, 0.0)
                         * jnp.maximum(yy2 - yy1, 0.0))
                iou = inter / (ka[sl] + area - inter + 1e-9)
                lanemask = (iota + kk * _L) < kc
                return jnp.maximum(acc, jnp.where(lanemask, iou, -1.0))

            nblk = (kc + (_L - 1)) // _L
            acc = lax.fori_loop(0, nblk, iou_blk,
                                jnp.full((_L,), -1.0, jnp.float32))
            suppressed = jnp.max(acc) > _IOU_THRESH
            valid = m > -1e8
            keep_it = valid & jnp.logical_not(suppressed)

            row = bx1
            row = jnp.where(iota == 1, by1, row)
            row = jnp.where(iota == 2, bx2, row)
            row = jnp.where(iota == 3, by2, row)
            row = jnp.where(iota == 4, m_v, row)

            @pl.when(keep_it)
            def _():
                kc_v = jnp.full((_L,), kc)
                plsc.store_scatter(kx1, [kc_v], bx1, mask=lane0)
                plsc.store_scatter(ky1, [kc_v], by1, mask=lane0)
                plsc.store_scatter(kx2, [kc_v], bx2, mask=lane0)
                plsc.store_scatter(ky2, [kc_v], by2, mask=lane0)
                plsc.store_scatter(ka, [kc_v], area, mask=lane0)
                plsc.store_scatter(out_ref, [kc_v * 5 + iota], row, mask=out5)

            filler_slot = jnp.logical_not(keep_it) & (fc < _MAX_DET)

            @pl.when(filler_slot)
            def _():
                fc_v = jnp.full((_L,), fc)
                plsc.store_scatter(fil_ref, [fc_v * 5 + iota], row, mask=out5)

            plsc.store_scatter(s_ref, [j_v], ninf, mask=lane0)
            sv_after = jnp.where(iota == j_v - c_v * _L, -jnp.inf, sv)
            l1_new = jnp.full((_L,), jnp.max(sv_after))
            plsc.store_scatter(l1_ref, [c_v], l1_new, mask=lane0)
            l1_after = jnp.where(iota == c_v - g_v * _L, l1_new, l1v)
            l2_new = jnp.full((_L,), jnp.max(l1_after))
            plsc.store_scatter(l2_ref, [g_v], l2_new, mask=lane0)
            l2_after = jnp.where(iota == g_v - q_v * _L, l2_new, l2v)
            l3_new = jnp.full((_L,), jnp.max(l2_after))
            plsc.store_scatter(l3_ref, [q_v], l3_new, mask=lane0)

            kc = jnp.where(keep_it, kc + 1, kc)
            fc = jnp.where(filler_slot, fc + 1, fc)
            return (kc, fc, processed + 1)

        kc, fc, _ = lax.while_loop(
            cond, body, (jnp.int32(0), jnp.int32(0), jnp.int32(0)))

        def fcond(i):
            return i < _MAX_DET

        def fbody(i):
            src = jnp.full((_L,), i - kc) * 5 + iota
            dst = jnp.full((_L,), i) * 5 + iota
            v = plsc.load_gather(fil_ref, [src])
            plsc.store_scatter(out_ref, [dst], v, mask=out5)
            return i + 1

        lax.while_loop(fcond, fbody, kc)

        pltpu.sync_copy(out_ref, out_hbm)


_sc_nms = functools.partial(
    pl.kernel,
    out_type=jax.ShapeDtypeStruct((_OUTW,), jnp.float32),
    mesh=plsc.VectorSubcoreMesh(core_axis_name="c", subcore_axis_name="s"),
    compiler_params=pltpu.CompilerParams(needs_layout_passes=False),
    scratch_types=[
        pltpu.VMEM((_C1 * _L,), jnp.float32),
        pltpu.VMEM((_N,), jnp.float32),
        pltpu.VMEM((_N,), jnp.float32),
        pltpu.VMEM((_N,), jnp.float32),
        pltpu.VMEM((_N,), jnp.float32),
        pltpu.VMEM((_C1 + _L,), jnp.float32),
        pltpu.VMEM((80,), jnp.float32),
        pltpu.VMEM((_L,), jnp.float32),
        pltpu.VMEM((_KCAP,), jnp.float32),
        pltpu.VMEM((_KCAP,), jnp.float32),
        pltpu.VMEM((_KCAP,), jnp.float32),
        pltpu.VMEM((_KCAP,), jnp.float32),
        pltpu.VMEM((_KCAP,), jnp.float32),
        pltpu.VMEM((_OUTW,), jnp.float32),
        pltpu.VMEM((_OUTW,), jnp.float32),
        pltpu.SemaphoreType.DMA,
    ],
)(_sc_body)


def kernel(boxes, scores):
    out = _sc_nms(boxes[:, 0], boxes[:, 1], boxes[:, 2], boxes[:, 3], scores)
    return out[:_MAX_DET * 5].reshape(_MAX_DET, 5)


# speedup vs baseline: 1.5561x; 1.0892x over previous
import functools

import jax
import jax.numpy as jnp
from jax import lax
from jax.experimental import pallas as pl
from jax.experimental.pallas import tpu as pltpu
from jax.experimental.pallas import tpu_sc as plsc

_N = 20000
_L = 16
_C1 = 1264
_MAX_DET = 100
_KCAP = 112
_OUTW = 512
_IOU_THRESH = 0.5
_SCORE_THRESH = 0.05
_NEG = -1e9


def _thresh(v):
    return jnp.where(v > _SCORE_THRESH, v,
                     jnp.where(v < -1e30, -jnp.inf, jnp.float32(_NEG)))


def _sc_body(x1_hbm, y1_hbm, x2_hbm, y2_hbm, sc_hbm, out_hbm,
             s_ref, x1_ref, y1_ref, x2_ref, y2_ref,
             l1_ref, l2_ref,
             kx1, ky1, kx2, ky2, ka, out_ref, fil_ref, dma_sem):
    tile0 = (lax.axis_index("c") == 0) & (lax.axis_index("s") == 0)

    @pl.when(tile0)
    def _():
        iota = lax.broadcasted_iota(jnp.int32, (_L,), 0)
        lane0 = iota == 0
        out5 = iota < 5
        ninf = jnp.full((_L,), -jnp.inf, jnp.float32)

        cps = [pltpu.async_copy(src, dst, dma_sem)
               for src, dst in ((x1_hbm, x1_ref), (y1_hbm, y1_ref),
                                (x2_hbm, x2_ref), (y2_hbm, y2_ref))]
        pltpu.sync_copy(sc_hbm, s_ref.at[pl.ds(0, _N)])

        for t in range((_C1 * _L - _N) // _L):
            s_ref[pl.ds(_N + t * _L, _L)] = ninf
        l1_ref[pl.ds(_C1, _L)] = ninf

        def l1_body(k, _):
            base = k * (_L * _L)
            acc = plsc.load_gather(s_ref, [base + iota * _L + iota])
            for t in range(1, _L):
                acc = jnp.maximum(
                    acc,
                    plsc.load_gather(
                        s_ref, [base + iota * _L + ((iota + t) & (_L - 1))]))
            l1_ref[pl.ds(k * _L, _L)] = _thresh(acc)
            return 0
        lax.fori_loop(0, _C1 // _L, l1_body, 0)

        def l2_body(k, _):
            base = k * (_L * _L)
            acc = plsc.load_gather(l1_ref, [base + iota * _L + iota])
            for t in range(1, _L):
                acc = jnp.maximum(
                    acc,
                    plsc.load_gather(
                        l1_ref, [base + iota * _L + ((iota + t) & (_L - 1))]))
            l2_ref[pl.ds(k * _L, _L)] = acc
            return 0
        lax.fori_loop(0, 80 // _L, l2_body, 0)

        l3_init = ninf
        for q in range(80 // _L):
            mq = jnp.max(l2_ref[pl.ds(q * _L, _L)])
            l3_init = jnp.where(iota == q, jnp.full((_L,), mq), l3_init)

        for cp in cps:
            cp.wait()

        def cond(st):
            kc, fc, processed = st[0], st[1], st[2]
            return (kc < _MAX_DET) & (processed < _N)

        def body(st):
            kc, fc, processed, l3v, m = st
            m_v = jnp.full((_L,), m)
            q_v = plsc.all_reduce_ffs(l3v == m_v)
            l2v = plsc.load_gather(l2_ref, [q_v * _L + iota])
            g_v = q_v * _L + plsc.all_reduce_ffs(l2v == m_v)
            l1v = plsc.load_gather(l1_ref, [g_v * _L + iota])
            c_v = g_v * _L + plsc.all_reduce_ffs(l1v == m_v)
            sv = _thresh(plsc.load_gather(s_ref, [c_v * _L + iota]))
            j_v = c_v * _L + plsc.all_reduce_ffs(sv == m_v)

            bx1 = plsc.load_gather(x1_ref, [j_v])
            by1 = plsc.load_gather(y1_ref, [j_v])
            bx2 = plsc.load_gather(x2_ref, [j_v])
            by2 = plsc.load_gather(y2_ref, [j_v])
            area = (jnp.maximum(bx2 - bx1, 0.0)
                    * jnp.maximum(by2 - by1, 0.0))

            def iou_blk(kk, acc):
                sl = pl.ds(kk * _L, _L)
                xx1 = jnp.maximum(kx1[sl], bx1)
                yy1 = jnp.maximum(ky1[sl], by1)
                xx2 = jnp.minimum(kx2[sl], bx2)
                yy2 = jnp.minimum(ky2[sl], by2)
                inter = (jnp.maximum(xx2 - xx1, 0.0)
                         * jnp.maximum(yy2 - yy1, 0.0))
                iou = inter / (ka[sl] + area - inter + 1e-9)
                lanemask = (iota + kk * _L) < kc
                return jnp.maximum(acc, jnp.where(lanemask, iou, -1.0))

            nblk = (kc + (_L - 1)) // _L
            acc = lax.fori_loop(0, nblk, iou_blk,
                                jnp.full((_L,), -1.0, jnp.float32))
            suppressed = jnp.max(acc) > _IOU_THRESH
            valid = m > -1e8
            keep_it = valid & jnp.logical_not(suppressed)

            row = bx1
            row = jnp.where(iota == 1, by1, row)
            row = jnp.where(iota == 2, bx2, row)
            row = jnp.where(iota == 3, by2, row)
            row = jnp.where(iota == 4, m_v, row)

            filler_slot = jnp.logical_not(keep_it) & (fc < _MAX_DET)
            kc_v = jnp.full((_L,), jnp.where(keep_it, kc, 101))
            fc_v = jnp.full((_L,), jnp.where(filler_slot, fc, 101))
            plsc.store_scatter(kx1, [kc_v], bx1, mask=lane0)
            plsc.store_scatter(ky1, [kc_v], by1, mask=lane0)
            plsc.store_scatter(kx2, [kc_v], bx2, mask=lane0)
            plsc.store_scatter(ky2, [kc_v], by2, mask=lane0)
            plsc.store_scatter(ka, [kc_v], area, mask=lane0)
            plsc.store_scatter(out_ref, [kc_v * 5 + iota], row, mask=out5)
            plsc.store_scatter(fil_ref, [fc_v * 5 + iota], row, mask=out5)

            plsc.store_scatter(s_ref, [j_v], ninf, mask=lane0)
            sv_after = jnp.where(iota == j_v - c_v * _L, -jnp.inf, sv)
            mx_ex_l1 = jnp.max(jnp.where(iota == c_v - g_v * _L, -jnp.inf,
                                         l1v))
            mx_ex_l2 = jnp.max(jnp.where(iota == g_v - q_v * _L, -jnp.inf,
                                         l2v))
            mx_ex_l3 = jnp.max(jnp.where(iota == q_v, -jnp.inf, l3v))
            l1_new = jnp.max(sv_after)
            plsc.store_scatter(l1_ref, [c_v], jnp.full((_L,), l1_new),
                               mask=lane0)
            l2_new = jnp.maximum(l1_new, mx_ex_l1)
            plsc.store_scatter(l2_ref, [g_v], jnp.full((_L,), l2_new),
                               mask=lane0)
            l3_new = jnp.maximum(l2_new, mx_ex_l2)
            l3v = jnp.where(iota == q_v, jnp.full((_L,), l3_new), l3v)
            m = jnp.maximum(l3_new, mx_ex_l3)

            kc = jnp.where(keep_it, kc + 1, kc)
            fc = jnp.where(filler_slot, fc + 1, fc)
            return (kc, fc, processed + 1, l3v, m)

        kc, fc, _, _, _ = lax.while_loop(
            cond, body, (jnp.int32(0), jnp.int32(0), jnp.int32(0),
                         l3_init, jnp.max(l3_init)))

        def fcond(i):
            return i < _MAX_DET

        def fbody(i):
            src = jnp.full((_L,), i - kc) * 5 + iota
            dst = jnp.full((_L,), i) * 5 + iota
            v = plsc.load_gather(fil_ref, [src])
            plsc.store_scatter(out_ref, [dst], v, mask=out5)
            return i + 1

        lax.while_loop(fcond, fbody, kc)

        pltpu.sync_copy(out_ref, out_hbm)


_sc_nms = functools.partial(
    pl.kernel,
    out_type=jax.ShapeDtypeStruct((_OUTW,), jnp.float32),
    mesh=plsc.VectorSubcoreMesh(core_axis_name="c", subcore_axis_name="s"),
    compiler_params=pltpu.CompilerParams(needs_layout_passes=False),
    scratch_types=[
        pltpu.VMEM((_C1 * _L,), jnp.float32),
        pltpu.VMEM((_N,), jnp.float32),
        pltpu.VMEM((_N,), jnp.float32),
        pltpu.VMEM((_N,), jnp.float32),
        pltpu.VMEM((_N,), jnp.float32),
        pltpu.VMEM((_C1 + _L,), jnp.float32),
        pltpu.VMEM((80,), jnp.float32),
        pltpu.VMEM((_KCAP,), jnp.float32),
        pltpu.VMEM((_KCAP,), jnp.float32),
        pltpu.VMEM((_KCAP,), jnp.float32),
        pltpu.VMEM((_KCAP,), jnp.float32),
        pltpu.VMEM((_KCAP,), jnp.float32),
        pltpu.VMEM((_OUTW,), jnp.float32),
        pltpu.VMEM((_OUTW,), jnp.float32),
        pltpu.SemaphoreType.DMA,
    ],
)(_sc_body)


def kernel(boxes, scores):
    out = _sc_nms(boxes[:, 0], boxes[:, 1], boxes[:, 2], boxes[:, 3], scores)
    return out[:_MAX_DET * 5].reshape(_MAX_DET, 5)
